# baseline (speedup 1.0000x reference)
import jax
import jax.numpy as jnp
from jax.experimental import pallas as pl
from jax.experimental.pallas import tpu as pltpu

DIM = 192
NUM_HEADS = 4
HEAD_DIM = 48
N_WIN = 28
RS = 8
NR = N_WIN * N_WIN
RSS = RS * RS
TOPK = 4
SCALE = DIM ** (-0.5)
ROWS = NR * RSS

GA = 56
RA = GA * RSS


def _qkv_body(x_ref, w_ref, b_ref, qkv_ref, qr_ref, kr_ref):
    y = jnp.dot(x_ref[:], w_ref[:], preferred_element_type=jnp.float32) + b_ref[0]
    qkv_ref[:] = y
    q = y[:, :DIM].reshape(GA, RSS, DIM)
    k = y[:, DIM:2 * DIM].reshape(GA, RSS, DIM)
    qr_ref[:] = jnp.mean(q, axis=1)
    kr_ref[:] = jnp.mean(k, axis=1)


def _qkv_proj(x2d, wT, b):
    return pl.pallas_call(
        _qkv_body,
        grid=(ROWS // RA,),
        in_specs=[
            pl.BlockSpec((RA, DIM), lambda i: (i, 0)),
            pl.BlockSpec((DIM, 3 * DIM), lambda i: (0, 0)),
            pl.BlockSpec((1, 3 * DIM), lambda i: (0, 0)),
        ],
        out_specs=[
            pl.BlockSpec((RA, 3 * DIM), lambda i: (i, 0)),
            pl.BlockSpec((GA, DIM), lambda i: (i, 0)),
            pl.BlockSpec((GA, DIM), lambda i: (i, 0)),
        ],
        out_shape=[
            jax.ShapeDtypeStruct((ROWS, 3 * DIM), jnp.float32),
            jax.ShapeDtypeStruct((NR, DIM), jnp.float32),
            jax.ShapeDtypeStruct((NR, DIM), jnp.float32),
        ],
    )(x2d, wT, b)


def _route_body(qr_ref, kr_ref, idx_ref):
    a = jax.lax.dot_general(qr_ref[:], kr_ref[:], (((1,), (1,)), ((), ())),
                            preferred_element_type=jnp.float32)
    iota = jax.lax.broadcasted_iota(jnp.int32, a.shape, 1)
    for t in range(TOPK):
        m = jnp.max(a, axis=1, keepdims=True)
        ii = jnp.min(jnp.where(a == m, iota, NR), axis=1)
        idx_ref[t] = ii
        a = jnp.where(iota == ii[:, None], -1e30, a)


def _route(qr, kr):
    return pl.pallas_call(
        _route_body,
        grid=(1,),
        in_specs=[
            pl.BlockSpec((NR, DIM), lambda i: (0, 0)),
            pl.BlockSpec((NR, DIM), lambda i: (0, 0)),
        ],
        out_specs=pl.BlockSpec((TOPK, NR), lambda i: (0, 0)),
        out_shape=jax.ShapeDtypeStruct((TOPK, NR), jnp.int32),
    )(qr, kr)


GC = 16


def _attn_body(idx_ref, q_ref, k_ref, v_ref, o_ref):
    rblk = pl.program_id(1)

    def per_region(r, carry):
        base = rblk * GC + r
        q = q_ref[0, r] * SCALE
        kg = jnp.concatenate(
            [k_ref[0, idx_ref[t, base]] for t in range(TOPK)],
            axis=0).astype(jnp.float32)
        vg = jnp.concatenate(
            [v_ref[0, idx_ref[t, base]] for t in range(TOPK)],
            axis=0).astype(jnp.float32)
        s = jax.lax.dot_general(q, kg, (((1,), (1,)), ((), ())),
                                preferred_element_type=jnp.float32)
        m = jnp.max(s, axis=1, keepdims=True)
        e = jnp.exp(s - m)
        p = e / jnp.sum(e, axis=1, keepdims=True)
        o_ref[0, r] = jnp.dot(p, vg, preferred_element_type=jnp.float32)
        return carry

    jax.lax.fori_loop(0, GC, per_region, 0)


def _attention(idx, qs, ks, vs):
    grid_spec = pltpu.PrefetchScalarGridSpec(
        num_scalar_prefetch=1,
        grid=(NUM_HEADS, NR // GC),
        in_specs=[
            pl.BlockSpec((1, GC, RSS, HEAD_DIM), lambda h, r, _: (h, r, 0, 0)),
            pl.BlockSpec((1, NR, RSS, HEAD_DIM), lambda h, r, _: (h, 0, 0, 0)),
            pl.BlockSpec((1, NR, RSS, HEAD_DIM), lambda h, r, _: (h, 0, 0, 0)),
        ],
        out_specs=pl.BlockSpec((1, GC, RSS, HEAD_DIM), lambda h, r, _: (h, r, 0, 0)),
    )
    return pl.pallas_call(
        _attn_body,
        grid_spec=grid_spec,
        out_shape=jax.ShapeDtypeStruct((NUM_HEADS, NR, RSS, HEAD_DIM), jnp.float32),
    )(idx, qs, ks, vs)


RCH = 8


def _lepe_body(va_ref, vb_ref, w_ref, b_ref, o_ref):
    rows10 = jnp.concatenate([va_ref[:], vb_ref[0:2]], axis=0)
    acc = jnp.zeros((RCH, 224, DIM), jnp.float32) + b_ref[0]
    for dy in range(3):
        for dx in range(3):
            acc = acc + rows10[dy:dy + RCH, dx:dx + 224, :] * w_ref[dy * 3 + dx]
    o_ref[:] = acc


def _lepe(vp, w9, b):
    return pl.pallas_call(
        _lepe_body,
        grid=(224 // RCH,),
        in_specs=[
            pl.BlockSpec((RCH, 226, DIM), lambda i: (i, 0, 0)),
            pl.BlockSpec((RCH, 226, DIM), lambda i: (i + 1, 0, 0)),
            pl.BlockSpec((9, DIM), lambda i: (0, 0)),
            pl.BlockSpec((1, DIM), lambda i: (0, 0)),
        ],
        out_specs=pl.BlockSpec((RCH, 224, DIM), lambda i: (i, 0, 0)),
        out_shape=jax.ShapeDtypeStruct((224, 224, DIM), jnp.float32),
    )(vp, vp, w9, b)


def _out_body(a_ref, l_ref, w_ref, b_ref, o_ref):
    o_ref[:] = jnp.dot(a_ref[:] + l_ref[:], w_ref[:],
                       preferred_element_type=jnp.float32) + b_ref[0]


def _out_proj(attn_sp, lepe2d, wT, b):
    return pl.pallas_call(
        _out_body,
        grid=(ROWS // RA,),
        in_specs=[
            pl.BlockSpec((RA, DIM), lambda i: (i, 0)),
            pl.BlockSpec((RA, DIM), lambda i: (i, 0)),
            pl.BlockSpec((DIM, DIM), lambda i: (0, 0)),
            pl.BlockSpec((1, DIM), lambda i: (0, 0)),
        ],
        out_specs=pl.BlockSpec((RA, DIM), lambda i: (i, 0)),
        out_shape=jax.ShapeDtypeStruct((ROWS, DIM), jnp.float32),
    )(attn_sp, lepe2d, wT, b)


def kernel(x, qkv_w, qkv_b, lepe_w, lepe_b, out_w, out_b):
    x2d = (x.reshape(N_WIN, RS, N_WIN, RS, DIM)
           .transpose(0, 2, 1, 3, 4)
           .reshape(ROWS, DIM))

    qkv, qr, kr = _qkv_proj(x2d, qkv_w.T, qkv_b.reshape(1, -1))
    idx = _route(qr, kr)

    def head_major(flat):
        return (flat.reshape(NR, RSS, NUM_HEADS, HEAD_DIM)
                .transpose(2, 0, 1, 3))

    qs = head_major(qkv[:, :DIM])
    ks = head_major(qkv[:, DIM:2 * DIM]).astype(jnp.bfloat16)
    vs = head_major(qkv[:, 2 * DIM:]).astype(jnp.bfloat16)
    attn = _attention(idx, qs, ks, vs)

    v_sp = (qkv[:, 2 * DIM:].reshape(N_WIN, N_WIN, RS, RS, DIM)
            .transpose(0, 2, 1, 3, 4)
            .reshape(224, 224, DIM))
    vp = jnp.pad(v_sp, ((1, 15), (1, 1), (0, 0)))
    lepe = _lepe(vp, lepe_w.reshape(DIM, 9).T, lepe_b.reshape(1, -1))

    attn_sp = (attn.transpose(1, 2, 0, 3)
               .reshape(N_WIN, N_WIN, RS, RS, DIM)
               .transpose(0, 2, 1, 3, 4)
               .reshape(ROWS, DIM))
    out = _out_proj(attn_sp, lepe.reshape(ROWS, DIM), out_w.T,
                    out_b.reshape(1, -1))
    return out.reshape(1, 224, 224, DIM)


# speedup vs baseline: 1.6417x; 1.6417x over previous
import jax
import jax.numpy as jnp
from jax.experimental import pallas as pl
from jax.experimental.pallas import tpu as pltpu

DIM = 192
NUM_HEADS = 4
HEAD_DIM = 48
N_WIN = 28
RS = 8
NR = N_WIN * N_WIN
RSS = RS * RS
TOPK = 4
SCALE = DIM ** (-0.5)
ROWS = NR * RSS
H = W = 224

RA = 3584


def _qkv_body(x_ref, w_ref, b_ref, q4_ref, k4_ref, v4_ref, vsp_ref,
              qr_ref, kr_ref):
    y = jnp.dot(x_ref[:], w_ref[:], preferred_element_type=jnp.float32) + b_ref[0]
    vsp_ref[:] = y[:, 2 * DIM:]
    y16 = y.astype(jnp.bfloat16)
    for h in range(NUM_HEADS):
        q4_ref[h] = y16[:, h * HEAD_DIM:(h + 1) * HEAD_DIM]
        k4_ref[h] = y16[:, DIM + h * HEAD_DIM:DIM + (h + 1) * HEAD_DIM]
        v4_ref[h] = y16[:, 2 * DIM + h * HEAD_DIM:2 * DIM + (h + 1) * HEAD_DIM]
    pooled = jnp.mean(y[:, :2 * DIM].reshape(2, RS, N_WIN, RS, 2 * DIM),
                      axis=(1, 3)).reshape(2 * N_WIN, 2 * DIM)
    qr_ref[:] = pooled[:, :DIM]
    kr_ref[:] = pooled[:, DIM:]


def _qkv_proj(x2d, wT, b):
    return pl.pallas_call(
        _qkv_body,
        grid=(ROWS // RA,),
        in_specs=[
            pl.BlockSpec((RA, DIM), lambda i: (i, 0)),
            pl.BlockSpec((DIM, 3 * DIM), lambda i: (0, 0)),
            pl.BlockSpec((1, 3 * DIM), lambda i: (0, 0)),
        ],
        out_specs=[
            pl.BlockSpec((NUM_HEADS, RA, HEAD_DIM), lambda i: (0, i, 0)),
            pl.BlockSpec((NUM_HEADS, RA, HEAD_DIM), lambda i: (0, i, 0)),
            pl.BlockSpec((NUM_HEADS, RA, HEAD_DIM), lambda i: (0, i, 0)),
            pl.BlockSpec((RA, DIM), lambda i: (i, 0)),
            pl.BlockSpec((2 * N_WIN, DIM), lambda i: (i, 0)),
            pl.BlockSpec((2 * N_WIN, DIM), lambda i: (i, 0)),
        ],
        out_shape=[
            jax.ShapeDtypeStruct((NUM_HEADS, ROWS, HEAD_DIM), jnp.bfloat16),
            jax.ShapeDtypeStruct((NUM_HEADS, ROWS, HEAD_DIM), jnp.bfloat16),
            jax.ShapeDtypeStruct((NUM_HEADS, ROWS, HEAD_DIM), jnp.bfloat16),
            jax.ShapeDtypeStruct((ROWS, DIM), jnp.float32),
            jax.ShapeDtypeStruct((NR, DIM), jnp.float32),
            jax.ShapeDtypeStruct((NR, DIM), jnp.float32),
        ],
    )(x2d, wT, b)


def _route_body(qr_ref, kr_ref, idx_ref):
    a = jax.lax.dot_general(qr_ref[:], kr_ref[:], (((1,), (1,)), ((), ())),
                            preferred_element_type=jnp.float32)
    iota = jax.lax.broadcasted_iota(jnp.int32, a.shape, 1)
    for t in range(TOPK):
        m = jnp.max(a, axis=1, keepdims=True)
        ii = jnp.min(jnp.where(a == m, iota, NR), axis=1)
        idx_ref[t] = ii
        a = jnp.where(iota == ii[:, None], -1e30, a)


def _route(qr, kr):
    return pl.pallas_call(
        _route_body,
        grid=(1,),
        in_specs=[
            pl.BlockSpec((NR, DIM), lambda i: (0, 0)),
            pl.BlockSpec((NR, DIM), lambda i: (0, 0)),
        ],
        out_specs=pl.BlockSpec((TOPK, NR), lambda i: (0, 0)),
        out_shape=jax.ShapeDtypeStruct((TOPK, NR), jnp.int32),
    )(qr, kr)


def _attn_body(idx_ref, q_ref, k_ref, v_ref, o_ref):
    i = pl.program_id(1)

    def region_tile(ref, jj):
        ji = jj // N_WIN
        jc = jj - ji * N_WIN
        t = ref[0, pl.ds(ji * RS, RS), pl.ds(jc * RS, RS), :]
        return t.reshape(RSS, HEAD_DIM)

    for j in range(N_WIN):
        r = i * N_WIN + j
        q = q_ref[0, :, j * RS:(j + 1) * RS, :].reshape(RSS, HEAD_DIM)
        kg = jnp.concatenate(
            [region_tile(k_ref, idx_ref[t, r]) for t in range(TOPK)], axis=0)
        vg = jnp.concatenate(
            [region_tile(v_ref, idx_ref[t, r]) for t in range(TOPK)], axis=0)
        s = jax.lax.dot_general(q, kg, (((1,), (1,)), ((), ())),
                                preferred_element_type=jnp.float32) * SCALE
        m = jnp.max(s, axis=1, keepdims=True)
        e = jnp.exp(s - m)
        p = (e / jnp.sum(e, axis=1, keepdims=True)).astype(jnp.bfloat16)
        o = jnp.dot(p, vg, preferred_element_type=jnp.float32)
        o_ref[0, :, j * RS:(j + 1) * RS, :] = o.reshape(RS, RS, HEAD_DIM).astype(jnp.bfloat16)


def _attention(idx, q4, k4, v4):
    grid_spec = pltpu.PrefetchScalarGridSpec(
        num_scalar_prefetch=1,
        grid=(NUM_HEADS, N_WIN),
        in_specs=[
            pl.BlockSpec((1, RS, W, HEAD_DIM), lambda h, i, _: (h, i, 0, 0)),
            pl.BlockSpec((1, H, W, HEAD_DIM), lambda h, i, _: (h, 0, 0, 0)),
            pl.BlockSpec((1, H, W, HEAD_DIM), lambda h, i, _: (h, 0, 0, 0)),
        ],
        out_specs=pl.BlockSpec((1, RS, W, HEAD_DIM), lambda h, i, _: (h, i, 0, 0)),
    )
    return pl.pallas_call(
        _attn_body,
        grid_spec=grid_spec,
        out_shape=jax.ShapeDtypeStruct((NUM_HEADS, H, W, HEAD_DIM), jnp.bfloat16),
    )(idx, q4, k4, v4)


RCH = 8


def _lepe_body(va_ref, vb_ref, w_ref, b_ref, o_ref):
    rows10 = jnp.concatenate([va_ref[:], vb_ref[0:2]], axis=0)
    acc = jnp.zeros((RCH, W, DIM), jnp.float32) + b_ref[0]
    for dy in range(3):
        for dx in range(3):
            acc = acc + rows10[dy:dy + RCH, dx:dx + W, :] * w_ref[dy * 3 + dx]
    o_ref[:] = acc


def _lepe(vp, w9, b):
    return pl.pallas_call(
        _lepe_body,
        grid=(H // RCH,),
        in_specs=[
            pl.BlockSpec((RCH, W + 2, DIM), lambda i: (i, 0, 0)),
            pl.BlockSpec((RCH, W + 2, DIM), lambda i: (i + 1, 0, 0)),
            pl.BlockSpec((9, DIM), lambda i: (0, 0)),
            pl.BlockSpec((1, DIM), lambda i: (0, 0)),
        ],
        out_specs=pl.BlockSpec((RCH, W, DIM), lambda i: (i, 0, 0)),
        out_shape=jax.ShapeDtypeStruct((H, W, DIM), jnp.float32),
    )(vp, vp, w9, b)


def _out_body(a_ref, l_ref, wh_ref, wl_ref, b_ref, o_ref):
    l16 = l_ref[:].astype(jnp.bfloat16).reshape(RCH * W, DIM)
    acc = jnp.dot(l16, wl_ref[:], preferred_element_type=jnp.float32)
    for h in range(NUM_HEADS):
        acc = acc + jnp.dot(a_ref[h].reshape(RCH * W, HEAD_DIM), wh_ref[h],
                            preferred_element_type=jnp.float32)
    o_ref[:] = (acc + b_ref[0]).reshape(RCH, W, DIM)


def _out_proj(attn4, lepe, wh, wl, b):
    return pl.pallas_call(
        _out_body,
        grid=(H // RCH,),
        in_specs=[
            pl.BlockSpec((NUM_HEADS, RCH, W, HEAD_DIM), lambda i: (0, i, 0, 0)),
            pl.BlockSpec((RCH, W, DIM), lambda i: (i, 0, 0)),
            pl.BlockSpec((NUM_HEADS, HEAD_DIM, DIM), lambda i: (0, 0, 0)),
            pl.BlockSpec((DIM, DIM), lambda i: (0, 0)),
            pl.BlockSpec((1, DIM), lambda i: (0, 0)),
        ],
        out_specs=pl.BlockSpec((RCH, W, DIM), lambda i: (i, 0, 0)),
        out_shape=jax.ShapeDtypeStruct((H, W, DIM), jnp.float32),
    )(attn4, lepe, wh, wl, b)


def kernel(x, qkv_w, qkv_b, lepe_w, lepe_b, out_w, out_b):
    x2d = x.reshape(ROWS, DIM)

    q4, k4, v4, v_sp, qr, kr = _qkv_proj(x2d, qkv_w.T, qkv_b.reshape(1, -1))
    idx = _route(qr, kr)

    q4 = q4.reshape(NUM_HEADS, H, W, HEAD_DIM)
    k4 = k4.reshape(NUM_HEADS, H, W, HEAD_DIM)
    v4 = v4.reshape(NUM_HEADS, H, W, HEAD_DIM)
    attn4 = _attention(idx, q4, k4, v4)

    vp = jnp.pad(v_sp.reshape(H, W, DIM), ((1, 15), (1, 1), (0, 0)))
    lepe = _lepe(vp, lepe_w.reshape(DIM, 9).T, lepe_b.reshape(1, -1))

    wT16 = out_w.T.astype(jnp.bfloat16)
    wh = wT16.reshape(NUM_HEADS, HEAD_DIM, DIM)
    out = _out_proj(attn4, lepe, wh, wT16, out_b.reshape(1, -1))
    return out.reshape(1, H, W, DIM)


# speedup vs baseline: 3.2663x; 1.9895x over previous
import jax
import jax.numpy as jnp
from jax.experimental import pallas as pl
from jax.experimental.pallas import tpu as pltpu

DIM = 192
NUM_HEADS = 4
HEAD_DIM = 48
N_WIN = 28
RS = 8
NR = N_WIN * N_WIN
RSS = RS * RS
TOPK = 4
SCALE = DIM ** (-0.5)
ROWS = NR * RSS
H = W = 224

RA = 3584


def _qkv_body(x_ref, w_ref, b_ref, q4_ref, k4_ref, v4_ref, vsp_ref,
              qr_ref, kr_ref):
    y = jnp.dot(x_ref[:], w_ref[:], preferred_element_type=jnp.float32) + b_ref[0]
    vsp_ref[:] = y[:, 2 * DIM:]
    y16 = y.astype(jnp.bfloat16)
    for h in range(NUM_HEADS):
        q4_ref[h] = y16[:, h * HEAD_DIM:(h + 1) * HEAD_DIM]
        k4_ref[h] = y16[:, DIM + h * HEAD_DIM:DIM + (h + 1) * HEAD_DIM]
        v4_ref[h] = y16[:, 2 * DIM + h * HEAD_DIM:2 * DIM + (h + 1) * HEAD_DIM]
    pooled = jnp.mean(y[:, :2 * DIM].reshape(2, RS, N_WIN, RS, 2 * DIM),
                      axis=(1, 3)).reshape(2 * N_WIN, 2 * DIM)
    qr_ref[:] = pooled[:, :DIM]
    kr_ref[:] = pooled[:, DIM:]


def _qkv_proj(x2d, wT, b):
    return pl.pallas_call(
        _qkv_body,
        grid=(ROWS // RA,),
        in_specs=[
            pl.BlockSpec((RA, DIM), lambda i: (i, 0)),
            pl.BlockSpec((DIM, 3 * DIM), lambda i: (0, 0)),
            pl.BlockSpec((1, 3 * DIM), lambda i: (0, 0)),
        ],
        out_specs=[
            pl.BlockSpec((NUM_HEADS, RA, HEAD_DIM), lambda i: (0, i, 0)),
            pl.BlockSpec((NUM_HEADS, RA, HEAD_DIM), lambda i: (0, i, 0)),
            pl.BlockSpec((NUM_HEADS, RA, HEAD_DIM), lambda i: (0, i, 0)),
            pl.BlockSpec((RA, DIM), lambda i: (i, 0)),
            pl.BlockSpec((2 * N_WIN, DIM), lambda i: (i, 0)),
            pl.BlockSpec((2 * N_WIN, DIM), lambda i: (i, 0)),
        ],
        out_shape=[
            jax.ShapeDtypeStruct((NUM_HEADS, ROWS, HEAD_DIM), jnp.bfloat16),
            jax.ShapeDtypeStruct((NUM_HEADS, ROWS, HEAD_DIM), jnp.bfloat16),
            jax.ShapeDtypeStruct((NUM_HEADS, ROWS, HEAD_DIM), jnp.bfloat16),
            jax.ShapeDtypeStruct((ROWS, DIM), jnp.float32),
            jax.ShapeDtypeStruct((NR, DIM), jnp.float32),
            jax.ShapeDtypeStruct((NR, DIM), jnp.float32),
        ],
    )(x2d, wT, b)


def _route_body(qr_ref, kr_ref, idx_ref):
    a = jax.lax.dot_general(qr_ref[:], kr_ref[:], (((1,), (1,)), ((), ())),
                            preferred_element_type=jnp.float32)
    iota = jax.lax.broadcasted_iota(jnp.int32, a.shape, 1)
    for t in range(TOPK):
        m = jnp.max(a, axis=1, keepdims=True)
        ii = jnp.min(jnp.where(a == m, iota, NR), axis=1)
        idx_ref[t] = ii
        a = jnp.where(iota == ii[:, None], -1e30, a)


def _route(qr, kr):
    return pl.pallas_call(
        _route_body,
        grid=(1,),
        in_specs=[
            pl.BlockSpec((NR, DIM), lambda i: (0, 0)),
            pl.BlockSpec((NR, DIM), lambda i: (0, 0)),
        ],
        out_specs=pl.BlockSpec((TOPK, NR), lambda i: (0, 0)),
        out_shape=jax.ShapeDtypeStruct((TOPK, NR), jnp.int32),
    )(qr, kr)


def _attn_body(idx_ref, q_ref, k_ref, v_ref, o_ref):
    i = pl.program_id(1)

    def region_tile(ref, jj):
        ji = jj // N_WIN
        jc = jj - ji * N_WIN
        t = ref[0, pl.ds(ji * RS, RS), pl.ds(jc * RS, RS), :]
        return t.reshape(RSS, HEAD_DIM)

    kgs, vgs = [], []
    for j in range(N_WIN):
        r = i * N_WIN + j
        kgs.append(jnp.concatenate(
            [region_tile(k_ref, idx_ref[t, r]) for t in range(TOPK)], axis=0))
        vgs.append(jnp.concatenate(
            [region_tile(v_ref, idx_ref[t, r]) for t in range(TOPK)], axis=0))
    KG = jnp.stack(kgs, axis=0)
    VG = jnp.stack(vgs, axis=0)
    Q = (q_ref[0].reshape(RS, N_WIN, RS, HEAD_DIM)
         .transpose(1, 0, 2, 3).reshape(N_WIN, RSS, HEAD_DIM))
    S = jax.lax.dot_general(Q, KG, (((2,), (2,)), ((0,), (0,))),
                            preferred_element_type=jnp.float32) * SCALE
    m = jnp.max(S, axis=2, keepdims=True)
    e = jnp.exp(S - m)
    P = (e / jnp.sum(e, axis=2, keepdims=True)).astype(jnp.bfloat16)
    O = jax.lax.dot_general(P, VG, (((2,), (1,)), ((0,), (0,))),
                            preferred_element_type=jnp.float32)
    o_ref[0] = (O.astype(jnp.bfloat16)
                .reshape(N_WIN, RS, RS, HEAD_DIM)
                .transpose(1, 0, 2, 3).reshape(RS, W, HEAD_DIM))


def _attention(idx, q4, k4, v4):
    grid_spec = pltpu.PrefetchScalarGridSpec(
        num_scalar_prefetch=1,
        grid=(NUM_HEADS, N_WIN),
        in_specs=[
            pl.BlockSpec((1, RS, W, HEAD_DIM), lambda h, i, _: (h, i, 0, 0)),
            pl.BlockSpec((1, H, W, HEAD_DIM), lambda h, i, _: (h, 0, 0, 0)),
            pl.BlockSpec((1, H, W, HEAD_DIM), lambda h, i, _: (h, 0, 0, 0)),
        ],
        out_specs=pl.BlockSpec((1, RS, W, HEAD_DIM), lambda h, i, _: (h, i, 0, 0)),
    )
    return pl.pallas_call(
        _attn_body,
        grid_spec=grid_spec,
        out_shape=jax.ShapeDtypeStruct((NUM_HEADS, H, W, HEAD_DIM), jnp.bfloat16),
    )(idx, q4, k4, v4)


RCH = 8


def _lepe_body(vp_ref, w_ref, b_ref, o_ref, buf, sem):
    i = pl.program_id(0)

    def dma(slot, blk):
        return pltpu.make_async_copy(vp_ref.at[pl.ds(blk * RCH, RCH + 2)],
                                     buf.at[slot], sem.at[slot])

    @pl.when(i == 0)
    def _():
        dma(0, 0).start()

    @pl.when(i + 1 < H // RCH)
    def _():
        dma((i + 1) % 2, i + 1).start()

    dma(i % 2, i).wait()
    rows10 = buf[i % 2]
    acc = jnp.zeros((RCH, W, DIM), jnp.float32) + b_ref[0]
    for dy in range(3):
        for dx in range(3):
            acc = acc + rows10[dy:dy + RCH, dx:dx + W, :] * w_ref[dy * 3 + dx]
    o_ref[:] = acc


def _lepe(vp, w9, b):
    return pl.pallas_call(
        _lepe_body,
        grid=(H // RCH,),
        in_specs=[
            pl.BlockSpec(memory_space=pl.ANY),
            pl.BlockSpec((9, DIM), lambda i: (0, 0)),
            pl.BlockSpec((1, DIM), lambda i: (0, 0)),
        ],
        out_specs=pl.BlockSpec((RCH, W, DIM), lambda i: (i, 0, 0)),
        out_shape=jax.ShapeDtypeStruct((H, W, DIM), jnp.float32),
        scratch_shapes=[
            pltpu.VMEM((2, RCH + 2, W + 2, DIM), jnp.float32),
            pltpu.SemaphoreType.DMA((2,)),
        ],
    )(vp, w9, b)


def _out_body(a_ref, l_ref, wh_ref, wl_ref, b_ref, o_ref):
    l16 = l_ref[:].astype(jnp.bfloat16).reshape(RCH * W, DIM)
    acc = jnp.dot(l16, wl_ref[:], preferred_element_type=jnp.float32)
    for h in range(NUM_HEADS):
        acc = acc + jnp.dot(a_ref[h].reshape(RCH * W, HEAD_DIM), wh_ref[h],
                            preferred_element_type=jnp.float32)
    o_ref[:] = (acc + b_ref[0]).reshape(RCH, W, DIM)


def _out_proj(attn4, lepe, wh, wl, b):
    return pl.pallas_call(
        _out_body,
        grid=(H // RCH,),
        in_specs=[
            pl.BlockSpec((NUM_HEADS, RCH, W, HEAD_DIM), lambda i: (0, i, 0, 0)),
            pl.BlockSpec((RCH, W, DIM), lambda i: (i, 0, 0)),
            pl.BlockSpec((NUM_HEADS, HEAD_DIM, DIM), lambda i: (0, 0, 0)),
            pl.BlockSpec((DIM, DIM), lambda i: (0, 0)),
            pl.BlockSpec((1, DIM), lambda i: (0, 0)),
        ],
        out_specs=pl.BlockSpec((RCH, W, DIM), lambda i: (i, 0, 0)),
        out_shape=jax.ShapeDtypeStruct((H, W, DIM), jnp.float32),
    )(attn4, lepe, wh, wl, b)


def kernel(x, qkv_w, qkv_b, lepe_w, lepe_b, out_w, out_b):
    x2d = x.reshape(ROWS, DIM)

    q4, k4, v4, v_sp, qr, kr = _qkv_proj(x2d, qkv_w.T, qkv_b.reshape(1, -1))
    idx = _route(qr, kr)

    q4 = q4.reshape(NUM_HEADS, H, W, HEAD_DIM)
    k4 = k4.reshape(NUM_HEADS, H, W, HEAD_DIM)
    v4 = v4.reshape(NUM_HEADS, H, W, HEAD_DIM)
    attn4 = _attention(idx, q4, k4, v4)

    vp = jnp.pad(v_sp.reshape(H, W, DIM), ((1, 15), (1, 1), (0, 0)))
    lepe = _lepe(vp, lepe_w.reshape(DIM, 9).T, lepe_b.reshape(1, -1))

    wT16 = out_w.T.astype(jnp.bfloat16)
    wh = wT16.reshape(NUM_HEADS, HEAD_DIM, DIM)
    out = _out_proj(attn4, lepe, wh, wT16, out_b.reshape(1, -1))
    return out.reshape(1, H, W, DIM)


# speedup vs baseline: 5.5087x; 1.6865x over previous
import jax
import jax.numpy as jnp
from jax.experimental import pallas as pl
from jax.experimental.pallas import tpu as pltpu

DIM = 192
NUM_HEADS = 4
HEAD_DIM = 48
N_WIN = 28
RS = 8
NR = N_WIN * N_WIN
RSS = RS * RS
TOPK = 4
SCALE = DIM ** (-0.5)
ROWS = NR * RSS
H = W = 224

RA = 3584


def _qkv_body(x_ref, w_ref, b_ref, q4_ref, k4_ref, v4_ref, vsp_ref,
              qr_ref, kr_ref):
    y = jax.lax.dot_general(x_ref[:], w_ref[:], (((1,), (0,)), ((), ())),
                            preferred_element_type=jnp.float32)
    y = y.reshape(RA, 3 * DIM) + b_ref[0]
    vsp_ref[:] = y[:, 2 * DIM:]
    y16 = y.astype(jnp.bfloat16)
    for h in range(NUM_HEADS):
        q4_ref[h] = y16[:, h * HEAD_DIM:(h + 1) * HEAD_DIM]
        k4_ref[h] = y16[:, DIM + h * HEAD_DIM:DIM + (h + 1) * HEAD_DIM]
        v4_ref[h] = y16[:, 2 * DIM + h * HEAD_DIM:2 * DIM + (h + 1) * HEAD_DIM]
    pooled = jnp.mean(y[:, :2 * DIM].reshape(2, RS, N_WIN, RS, 2 * DIM),
                      axis=(1, 3)).reshape(2 * N_WIN, 2 * DIM)
    qr_ref[:] = pooled[:, :DIM]
    kr_ref[:] = pooled[:, DIM:]


def _qkv_proj(x2d, wT, b):
    return pl.pallas_call(
        _qkv_body,
        grid=(ROWS // RA,),
        in_specs=[
            pl.BlockSpec((RA // W, DIM, W), lambda i: (i, 0, 0)),
            pl.BlockSpec((DIM, 3 * DIM), lambda i: (0, 0)),
            pl.BlockSpec((1, 3 * DIM), lambda i: (0, 0)),
        ],
        out_specs=[
            pl.BlockSpec((NUM_HEADS, RA, HEAD_DIM), lambda i: (0, i, 0)),
            pl.BlockSpec((NUM_HEADS, RA, HEAD_DIM), lambda i: (0, i, 0)),
            pl.BlockSpec((NUM_HEADS, RA, HEAD_DIM), lambda i: (0, i, 0)),
            pl.BlockSpec((RA, DIM), lambda i: (i, 0)),
            pl.BlockSpec((2 * N_WIN, DIM), lambda i: (i, 0)),
            pl.BlockSpec((2 * N_WIN, DIM), lambda i: (i, 0)),
        ],
        out_shape=[
            jax.ShapeDtypeStruct((NUM_HEADS, ROWS, HEAD_DIM), jnp.bfloat16),
            jax.ShapeDtypeStruct((NUM_HEADS, ROWS, HEAD_DIM), jnp.bfloat16),
            jax.ShapeDtypeStruct((NUM_HEADS, ROWS, HEAD_DIM), jnp.bfloat16),
            jax.ShapeDtypeStruct((ROWS, DIM), jnp.float32),
            jax.ShapeDtypeStruct((NR, DIM), jnp.float32),
            jax.ShapeDtypeStruct((NR, DIM), jnp.float32),
        ],
    )(x2d, wT, b)


def _route_body(qr_ref, kr_ref, idx_ref):
    a = jax.lax.dot_general(qr_ref[:], kr_ref[:], (((1,), (1,)), ((), ())),
                            preferred_element_type=jnp.float32)
    iota = jax.lax.broadcasted_iota(jnp.int32, a.shape, 1)
    for t in range(TOPK):
        m = jnp.max(a, axis=1, keepdims=True)
        ii = jnp.min(jnp.where(a == m, iota, NR), axis=1)
        idx_ref[t] = ii
        a = jnp.where(iota == ii[:, None], -1e30, a)


def _route(qr, kr):
    return pl.pallas_call(
        _route_body,
        grid=(1,),
        in_specs=[
            pl.BlockSpec((NR, DIM), lambda i: (0, 0)),
            pl.BlockSpec((NR, DIM), lambda i: (0, 0)),
        ],
        out_specs=pl.BlockSpec((TOPK, NR), lambda i: (0, 0)),
        out_shape=jax.ShapeDtypeStruct((TOPK, NR), jnp.int32),
    )(qr, kr)


def _attn_body(idx_ref, q_ref, k_ref, v_ref, o_ref):
    i = pl.program_id(1)

    def region_tile(ref, jj):
        ji = jj // N_WIN
        jc = jj - ji * N_WIN
        t = ref[0, pl.ds(ji * RS, RS), pl.ds(jc * RS, RS), :]
        return t.reshape(RSS, HEAD_DIM)

    kgs, vgs = [], []
    for j in range(N_WIN):
        r = i * N_WIN + j
        kgs.append(jnp.concatenate(
            [region_tile(k_ref, idx_ref[t, r]) for t in range(TOPK)], axis=0))
        vgs.append(jnp.concatenate(
            [region_tile(v_ref, idx_ref[t, r]) for t in range(TOPK)], axis=0))
    KG = jnp.stack(kgs, axis=0)
    VG = jnp.stack(vgs, axis=0)
    Q = (q_ref[0].reshape(RS, N_WIN, RS, HEAD_DIM)
         .transpose(1, 0, 2, 3).reshape(N_WIN, RSS, HEAD_DIM))
    S = jax.lax.dot_general(Q, KG, (((2,), (2,)), ((0,), (0,))),
                            preferred_element_type=jnp.float32) * SCALE
    m = jnp.max(S, axis=2, keepdims=True)
    e = jnp.exp(S - m)
    P = (e / jnp.sum(e, axis=2, keepdims=True)).astype(jnp.bfloat16)
    O = jax.lax.dot_general(P, VG, (((2,), (1,)), ((0,), (0,))),
                            preferred_element_type=jnp.float32)
    o_ref[0] = (O.astype(jnp.bfloat16)
                .reshape(N_WIN, RS, RS, HEAD_DIM)
                .transpose(1, 0, 2, 3).reshape(RS, W, HEAD_DIM))


def _attention(idx, q4, k4, v4):
    grid_spec = pltpu.PrefetchScalarGridSpec(
        num_scalar_prefetch=1,
        grid=(NUM_HEADS, N_WIN),
        in_specs=[
            pl.BlockSpec((1, RS, W, HEAD_DIM), lambda h, i, _: (h, i, 0, 0)),
            pl.BlockSpec((1, H, W, HEAD_DIM), lambda h, i, _: (h, 0, 0, 0)),
            pl.BlockSpec((1, H, W, HEAD_DIM), lambda h, i, _: (h, 0, 0, 0)),
        ],
        out_specs=pl.BlockSpec((1, RS, W, HEAD_DIM), lambda h, i, _: (h, i, 0, 0)),
    )
    return pl.pallas_call(
        _attn_body,
        grid_spec=grid_spec,
        out_shape=jax.ShapeDtypeStruct((NUM_HEADS, H, W, HEAD_DIM), jnp.bfloat16),
    )(idx, q4, k4, v4)


RCH = 8


def _lepe_body(vp_ref, w_ref, b_ref, o_ref, buf, sem):
    i = pl.program_id(0)

    def dma(slot, blk):
        return pltpu.make_async_copy(vp_ref.at[pl.ds(blk * RCH, RCH + 2)],
                                     buf.at[slot], sem.at[slot])

    @pl.when(i == 0)
    def _():
        dma(0, 0).start()

    @pl.when(i + 1 < H // RCH)
    def _():
        dma((i + 1) % 2, i + 1).start()

    dma(i % 2, i).wait()
    rows10 = buf[i % 2]
    acc = jnp.zeros((RCH, W, DIM), jnp.float32) + b_ref[0]
    for dy in range(3):
        for dx in range(3):
            acc = acc + rows10[dy:dy + RCH, dx:dx + W, :] * w_ref[dy * 3 + dx]
    o_ref[:] = acc


def _lepe(vp, w9, b):
    return pl.pallas_call(
        _lepe_body,
        grid=(H // RCH,),
        in_specs=[
            pl.BlockSpec(memory_space=pl.ANY),
            pl.BlockSpec((9, DIM), lambda i: (0, 0)),
            pl.BlockSpec((1, DIM), lambda i: (0, 0)),
        ],
        out_specs=pl.BlockSpec((RCH, W, DIM), lambda i: (i, 0, 0)),
        out_shape=jax.ShapeDtypeStruct((H, W, DIM), jnp.float32),
        scratch_shapes=[
            pltpu.VMEM((2, RCH + 2, W + 2, DIM), jnp.float32),
            pltpu.SemaphoreType.DMA((2,)),
        ],
    )(vp, w9, b)


def _out_body(a_ref, l_ref, wh_ref, wl_ref, b_ref, o_ref):
    l16 = l_ref[:].astype(jnp.bfloat16)
    for h in range(RCH):
        acc = jax.lax.dot_general(wl_ref[:], l16[h], (((0,), (1,)), ((), ())),
                                  preferred_element_type=jnp.float32)
        for hd in range(NUM_HEADS):
            acc = acc + jax.lax.dot_general(
                wh_ref[hd], a_ref[hd, h], (((0,), (1,)), ((), ())),
                preferred_element_type=jnp.float32)
        o_ref[h] = acc + b_ref[:]


def _out_proj(attn4, lepe, wh, wl, b):
    return pl.pallas_call(
        _out_body,
        grid=(H // RCH,),
        in_specs=[
            pl.BlockSpec((NUM_HEADS, RCH, W, HEAD_DIM), lambda i: (0, i, 0, 0)),
            pl.BlockSpec((RCH, W, DIM), lambda i: (i, 0, 0)),
            pl.BlockSpec((NUM_HEADS, HEAD_DIM, DIM), lambda i: (0, 0, 0)),
            pl.BlockSpec((DIM, DIM), lambda i: (0, 0)),
            pl.BlockSpec((DIM, 1), lambda i: (0, 0)),
        ],
        out_specs=pl.BlockSpec((RCH, DIM, W), lambda i: (i, 0, 0)),
        out_shape=jax.ShapeDtypeStruct((H, DIM, W), jnp.float32),
    )(attn4, lepe, wh, wl, b)


def kernel(x, qkv_w, qkv_b, lepe_w, lepe_b, out_w, out_b):
    x_t = jnp.transpose(x, (0, 1, 3, 2)).reshape(H, DIM, W)

    q4, k4, v4, v_sp, qr, kr = _qkv_proj(x_t, qkv_w.T, qkv_b.reshape(1, -1))
    idx = _route(qr, kr)

    q4 = q4.reshape(NUM_HEADS, H, W, HEAD_DIM)
    k4 = k4.reshape(NUM_HEADS, H, W, HEAD_DIM)
    v4 = v4.reshape(NUM_HEADS, H, W, HEAD_DIM)
    attn4 = _attention(idx, q4, k4, v4)

    vp = jnp.pad(v_sp.reshape(H, W, DIM), ((1, 15), (1, 1), (0, 0)))
    lepe = _lepe(vp, lepe_w.reshape(DIM, 9).T, lepe_b.reshape(1, -1))

    wT16 = out_w.T.astype(jnp.bfloat16)
    wh = wT16.reshape(NUM_HEADS, HEAD_DIM, DIM)
    out = _out_proj(attn4, lepe, wh, wT16, out_b.reshape(-1, 1))
    return jnp.transpose(out.reshape(1, H, DIM, W), (0, 1, 3, 2))


# speedup vs baseline: 6.4871x; 1.1776x over previous
import jax
import jax.numpy as jnp
from jax.experimental import pallas as pl
from jax.experimental.pallas import tpu as pltpu

DIM = 192
NUM_HEADS = 4
HEAD_DIM = 48
N_WIN = 28
RS = 8
NR = N_WIN * N_WIN
RSS = RS * RS
TOPK = 4
SCALE = DIM ** (-0.5)
ROWS = NR * RSS
H = W = 224

RA = 3584


def _qkv_body(x_ref, w_ref, b_ref, q4_ref, k4_ref, v4_ref, vsp_ref,
              qr_ref, kr_ref):
    y = jax.lax.dot_general(x_ref[:], w_ref[:], (((1,), (0,)), ((), ())),
                            preferred_element_type=jnp.float32)
    y = y.reshape(RA, 3 * DIM) + b_ref[0]
    y16 = y.astype(jnp.bfloat16)
    vsp_ref[:] = y16[:, 2 * DIM:]
    for h in range(NUM_HEADS):
        q4_ref[h] = y16[:, h * HEAD_DIM:(h + 1) * HEAD_DIM]
        k4_ref[h] = y16[:, DIM + h * HEAD_DIM:DIM + (h + 1) * HEAD_DIM]
        v4_ref[h] = y16[:, 2 * DIM + h * HEAD_DIM:2 * DIM + (h + 1) * HEAD_DIM]
    pooled = jnp.mean(y[:, :2 * DIM].reshape(2, RS, N_WIN, RS, 2 * DIM),
                      axis=(1, 3)).reshape(2 * N_WIN, 2 * DIM)
    qr_ref[:] = pooled[:, :DIM]
    kr_ref[:] = pooled[:, DIM:]


def _qkv_proj(x2d, wT, b):
    return pl.pallas_call(
        _qkv_body,
        grid=(ROWS // RA,),
        in_specs=[
            pl.BlockSpec((RA // W, DIM, W), lambda i: (i, 0, 0)),
            pl.BlockSpec((DIM, 3 * DIM), lambda i: (0, 0)),
            pl.BlockSpec((1, 3 * DIM), lambda i: (0, 0)),
        ],
        out_specs=[
            pl.BlockSpec((NUM_HEADS, RA, HEAD_DIM), lambda i: (0, i, 0)),
            pl.BlockSpec((NUM_HEADS, RA, HEAD_DIM), lambda i: (0, i, 0)),
            pl.BlockSpec((NUM_HEADS, RA, HEAD_DIM), lambda i: (0, i, 0)),
            pl.BlockSpec((RA, DIM), lambda i: (i, 0)),
            pl.BlockSpec((2 * N_WIN, DIM), lambda i: (i, 0)),
            pl.BlockSpec((2 * N_WIN, DIM), lambda i: (i, 0)),
        ],
        out_shape=[
            jax.ShapeDtypeStruct((NUM_HEADS, ROWS, HEAD_DIM), jnp.bfloat16),
            jax.ShapeDtypeStruct((NUM_HEADS, ROWS, HEAD_DIM), jnp.bfloat16),
            jax.ShapeDtypeStruct((NUM_HEADS, ROWS, HEAD_DIM), jnp.bfloat16),
            jax.ShapeDtypeStruct((ROWS, DIM), jnp.bfloat16),
            jax.ShapeDtypeStruct((NR, DIM), jnp.float32),
            jax.ShapeDtypeStruct((NR, DIM), jnp.float32),
        ],
    )(x2d, wT, b)


def _route_body(qr_ref, kr_ref, ri_ref, rc_ref):
    a = jax.lax.dot_general(qr_ref[:], kr_ref[:], (((1,), (1,)), ((), ())),
                            preferred_element_type=jnp.float32)
    iota = jax.lax.broadcasted_iota(jnp.int32, a.shape, 1)
    for t in range(TOPK):
        m = jnp.max(a, axis=1, keepdims=True)
        ii = jnp.min(jnp.where(a == m, iota, NR), axis=1)
        ji = ii // N_WIN
        ri_ref[t] = ji * RS
        rc_ref[t] = (ii - ji * N_WIN) * RS
        a = jnp.where(iota == ii[:, None], -1e30, a)


def _route(qr, kr):
    return pl.pallas_call(
        _route_body,
        grid=(1,),
        in_specs=[
            pl.BlockSpec((NR, DIM), lambda i: (0, 0)),
            pl.BlockSpec((NR, DIM), lambda i: (0, 0)),
        ],
        out_specs=[
            pl.BlockSpec((TOPK, NR), lambda i: (0, 0)),
            pl.BlockSpec((TOPK, NR), lambda i: (0, 0)),
        ],
        out_shape=[
            jax.ShapeDtypeStruct((TOPK, NR), jnp.int32),
            jax.ShapeDtypeStruct((TOPK, NR), jnp.int32),
        ],
    )(qr, kr)


def _attn_body(ri_ref, rc_ref, q_ref, k_ref, v_ref, o_ref):
    i = pl.program_id(1)

    def region_tile(ref, ro, co):
        t = ref[0, pl.ds(ro, RS), pl.ds(co, RS), :]
        return t.reshape(RSS, HEAD_DIM)

    kgs, vgs = [], []
    for j in range(N_WIN):
        r = i * N_WIN + j
        offs = [(pl.multiple_of(ri_ref[t, r], RS),
                 pl.multiple_of(rc_ref[t, r], RS)) for t in range(TOPK)]
        kgs.append(jnp.concatenate(
            [region_tile(k_ref, ro, co) for ro, co in offs], axis=0))
        vgs.append(jnp.concatenate(
            [region_tile(v_ref, ro, co) for ro, co in offs], axis=0))
    KG = jnp.stack(kgs, axis=0)
    VG = jnp.stack(vgs, axis=0)
    Q = (q_ref[0].reshape(RS, N_WIN, RS, HEAD_DIM)
         .transpose(1, 0, 2, 3).reshape(N_WIN, RSS, HEAD_DIM))
    S = jax.lax.dot_general(Q, KG, (((2,), (2,)), ((0,), (0,))),
                            preferred_element_type=jnp.float32) * SCALE
    m = jnp.max(S, axis=2, keepdims=True)
    e = jnp.exp(S - m)
    P = (e / jnp.sum(e, axis=2, keepdims=True)).astype(jnp.bfloat16)
    O = jax.lax.dot_general(P, VG, (((2,), (1,)), ((0,), (0,))),
                            preferred_element_type=jnp.float32)
    o_ref[0] = (O.astype(jnp.bfloat16)
                .reshape(N_WIN, RS, RS, HEAD_DIM)
                .transpose(1, 0, 2, 3).reshape(RS, W, HEAD_DIM))


def _attention(ri, rc, q4, k4, v4):
    grid_spec = pltpu.PrefetchScalarGridSpec(
        num_scalar_prefetch=2,
        grid=(NUM_HEADS, N_WIN),
        in_specs=[
            pl.BlockSpec((1, RS, W, HEAD_DIM), lambda h, i, *_: (h, i, 0, 0)),
            pl.BlockSpec((1, H, W, HEAD_DIM), lambda h, i, *_: (h, 0, 0, 0)),
            pl.BlockSpec((1, H, W, HEAD_DIM), lambda h, i, *_: (h, 0, 0, 0)),
        ],
        out_specs=pl.BlockSpec((1, RS, W, HEAD_DIM), lambda h, i, *_: (h, i, 0, 0)),
    )
    return pl.pallas_call(
        _attn_body,
        grid_spec=grid_spec,
        out_shape=jax.ShapeDtypeStruct((NUM_HEADS, H, W, HEAD_DIM), jnp.bfloat16),
    )(ri, rc, q4, k4, v4)


RCH = 8


def _out_body(a_ref, vp_ref, wh_ref, wl_ref, w9_ref, lb_ref, b_ref, o_ref,
              buf, sem):
    i = pl.program_id(0)

    def dma(slot, blk):
        return pltpu.make_async_copy(vp_ref.at[pl.ds(blk * RCH, RCH + 2)],
                                     buf.at[slot], sem.at[slot])

    @pl.when(i == 0)
    def _():
        dma(0, 0).start()

    @pl.when(i + 1 < H // RCH)
    def _():
        dma((i + 1) % 2, i + 1).start()

    dma(i % 2, i).wait()
    rows10 = buf[i % 2].astype(jnp.float32)
    lep = jnp.zeros((RCH, W, DIM), jnp.float32) + lb_ref[0]
    for dy in range(3):
        for dx in range(3):
            lep = lep + rows10[dy:dy + RCH, dx:dx + W, :] * w9_ref[dy * 3 + dx]
    l16 = lep.astype(jnp.bfloat16)
    for h in range(RCH):
        acc = jax.lax.dot_general(wl_ref[:], l16[h], (((0,), (1,)), ((), ())),
                                  preferred_element_type=jnp.float32)
        for hd in range(NUM_HEADS):
            acc = acc + jax.lax.dot_general(
                wh_ref[hd], a_ref[hd, h], (((0,), (1,)), ((), ())),
                preferred_element_type=jnp.float32)
        o_ref[h] = acc + b_ref[:]


def _out_proj(attn4, vp, wh, wl, w9, lb, b):
    return pl.pallas_call(
        _out_body,
        grid=(H // RCH,),
        in_specs=[
            pl.BlockSpec((NUM_HEADS, RCH, W, HEAD_DIM), lambda i: (0, i, 0, 0)),
            pl.BlockSpec(memory_space=pl.ANY),
            pl.BlockSpec((NUM_HEADS, HEAD_DIM, DIM), lambda i: (0, 0, 0)),
            pl.BlockSpec((DIM, DIM), lambda i: (0, 0)),
            pl.BlockSpec((9, DIM), lambda i: (0, 0)),
            pl.BlockSpec((1, DIM), lambda i: (0, 0)),
            pl.BlockSpec((DIM, 1), lambda i: (0, 0)),
        ],
        out_specs=pl.BlockSpec((RCH, DIM, W), lambda i: (i, 0, 0)),
        out_shape=jax.ShapeDtypeStruct((H, DIM, W), jnp.float32),
        scratch_shapes=[
            pltpu.VMEM((2, RCH + 2, W + 2, DIM), jnp.bfloat16),
            pltpu.SemaphoreType.DMA((2,)),
        ],
    )(attn4, vp, wh, wl, w9, lb, b)


def kernel(x, qkv_w, qkv_b, lepe_w, lepe_b, out_w, out_b):
    x_t = jnp.transpose(x, (0, 1, 3, 2)).reshape(H, DIM, W)

    q4, k4, v4, v_sp, qr, kr = _qkv_proj(x_t, qkv_w.T, qkv_b.reshape(1, -1))
    ri, rc = _route(qr, kr)

    q4 = q4.reshape(NUM_HEADS, H, W, HEAD_DIM)
    k4 = k4.reshape(NUM_HEADS, H, W, HEAD_DIM)
    v4 = v4.reshape(NUM_HEADS, H, W, HEAD_DIM)
    attn4 = _attention(ri, rc, q4, k4, v4)

    vp = jnp.pad(v_sp.reshape(H, W, DIM), ((1, 15), (1, 1), (0, 0)))

    wT16 = out_w.T.astype(jnp.bfloat16)
    wh = wT16.reshape(NUM_HEADS, HEAD_DIM, DIM)
    out = _out_proj(attn4, vp, wh, wT16, lepe_w.reshape(DIM, 9).T,
                    lepe_b.reshape(1, -1), out_b.reshape(-1, 1))
    return jnp.transpose(out.reshape(1, H, DIM, W), (0, 1, 3, 2))


# speedup vs baseline: 6.7327x; 1.0379x over previous
import jax
import jax.numpy as jnp
from jax.experimental import pallas as pl
from jax.experimental.pallas import tpu as pltpu

DIM = 192
NUM_HEADS = 4
HEAD_DIM = 48
N_WIN = 28
RS = 8
NR = N_WIN * N_WIN
RSS = RS * RS
TOPK = 4
SCALE = DIM ** (-0.5)
ROWS = NR * RSS
H = W = 224

RA = 3584


def _qkv_body(x_ref, w_ref, b_ref, q4_ref, k4_ref, v4_ref, vsp_ref,
              qr_ref, kr_ref):
    y = jax.lax.dot_general(x_ref[:], w_ref[:], (((1,), (0,)), ((), ())),
                            preferred_element_type=jnp.float32)
    y = y.reshape(RA, 3 * DIM) + b_ref[0]
    y16 = y.astype(jnp.bfloat16)
    vsp_ref[:] = y16[:, 2 * DIM:]
    yq = (y[:, :DIM] * SCALE).astype(jnp.bfloat16)
    for h in range(NUM_HEADS):
        q4_ref[h] = yq[:, h * HEAD_DIM:(h + 1) * HEAD_DIM]
        k4_ref[h] = y16[:, DIM + h * HEAD_DIM:DIM + (h + 1) * HEAD_DIM]
        v4_ref[h] = y16[:, 2 * DIM + h * HEAD_DIM:2 * DIM + (h + 1) * HEAD_DIM]
    pooled = jnp.mean(y[:, :2 * DIM].reshape(2, RS, N_WIN, RS, 2 * DIM),
                      axis=(1, 3)).reshape(2 * N_WIN, 2 * DIM)
    qr_ref[:] = pooled[:, :DIM]
    kr_ref[:] = pooled[:, DIM:]


def _qkv_proj(x2d, wT, b):
    return pl.pallas_call(
        _qkv_body,
        grid=(ROWS // RA,),
        in_specs=[
            pl.BlockSpec((RA // W, DIM, W), lambda i: (i, 0, 0)),
            pl.BlockSpec((DIM, 3 * DIM), lambda i: (0, 0)),
            pl.BlockSpec((1, 3 * DIM), lambda i: (0, 0)),
        ],
        out_specs=[
            pl.BlockSpec((NUM_HEADS, RA, HEAD_DIM), lambda i: (0, i, 0)),
            pl.BlockSpec((NUM_HEADS, RA, HEAD_DIM), lambda i: (0, i, 0)),
            pl.BlockSpec((NUM_HEADS, RA, HEAD_DIM), lambda i: (0, i, 0)),
            pl.BlockSpec((RA, DIM), lambda i: (i, 0)),
            pl.BlockSpec((2 * N_WIN, DIM), lambda i: (i, 0)),
            pl.BlockSpec((2 * N_WIN, DIM), lambda i: (i, 0)),
        ],
        out_shape=[
            jax.ShapeDtypeStruct((NUM_HEADS, ROWS, HEAD_DIM), jnp.bfloat16),
            jax.ShapeDtypeStruct((NUM_HEADS, ROWS, HEAD_DIM), jnp.bfloat16),
            jax.ShapeDtypeStruct((NUM_HEADS, ROWS, HEAD_DIM), jnp.bfloat16),
            jax.ShapeDtypeStruct((ROWS, DIM), jnp.bfloat16),
            jax.ShapeDtypeStruct((NR, DIM), jnp.float32),
            jax.ShapeDtypeStruct((NR, DIM), jnp.float32),
        ],
    )(x2d, wT, b)


def _route_body(qr_ref, kr_ref, ri_ref, rc_ref):
    a = jax.lax.dot_general(qr_ref[:], kr_ref[:], (((1,), (1,)), ((), ())),
                            preferred_element_type=jnp.float32)
    iota = jax.lax.broadcasted_iota(jnp.int32, a.shape, 1)
    for t in range(TOPK):
        m = jnp.max(a, axis=1, keepdims=True)
        ii = jnp.min(jnp.where(a == m, iota, NR), axis=1)
        ji = ii // N_WIN
        ri_ref[t] = ji * RS
        rc_ref[t] = (ii - ji * N_WIN) * RS
        a = jnp.where(iota == ii[:, None], -1e30, a)


def _route(qr, kr):
    return pl.pallas_call(
        _route_body,
        grid=(1,),
        in_specs=[
            pl.BlockSpec((NR, DIM), lambda i: (0, 0)),
            pl.BlockSpec((NR, DIM), lambda i: (0, 0)),
        ],
        out_specs=[
            pl.BlockSpec((TOPK, NR), lambda i: (0, 0)),
            pl.BlockSpec((TOPK, NR), lambda i: (0, 0)),
        ],
        out_shape=[
            jax.ShapeDtypeStruct((TOPK, NR), jnp.int32),
            jax.ShapeDtypeStruct((TOPK, NR), jnp.int32),
        ],
    )(qr, kr)


def _attn_body(ri_ref, rc_ref, q_ref, k_ref, v_ref, o_ref):
    i = pl.program_id(1)

    def region_tile(ref, ro, co):
        t = ref[0, pl.ds(ro, RS), pl.ds(co, RS), :]
        return t.reshape(RSS, HEAD_DIM)

    kgs, vgs = [], []
    for j in range(N_WIN):
        r = i * N_WIN + j
        offs = [(pl.multiple_of(ri_ref[t, r], RS),
                 pl.multiple_of(rc_ref[t, r], RS)) for t in range(TOPK)]
        kgs.append(jnp.concatenate(
            [region_tile(k_ref, ro, co) for ro, co in offs], axis=0))
        vgs.append(jnp.concatenate(
            [region_tile(v_ref, ro, co) for ro, co in offs], axis=0))
    KG = jnp.stack(kgs, axis=0)
    VG = jnp.stack(vgs, axis=0)
    Q = (q_ref[0].reshape(RS, N_WIN, RS, HEAD_DIM)
         .transpose(1, 0, 2, 3).reshape(N_WIN, RSS, HEAD_DIM))
    S = jax.lax.dot_general(Q, KG, (((2,), (2,)), ((0,), (0,))),
                            preferred_element_type=jnp.float32)
    m = jnp.max(S, axis=2, keepdims=True)
    e = jnp.exp(S - m)
    ssum = jnp.sum(e, axis=2, keepdims=True)
    O = jax.lax.dot_general(e.astype(jnp.bfloat16), VG,
                            (((2,), (1,)), ((0,), (0,))),
                            preferred_element_type=jnp.float32) / ssum
    o_ref[0] = (O.astype(jnp.bfloat16)
                .reshape(N_WIN, RS, RS, HEAD_DIM)
                .transpose(1, 0, 2, 3).reshape(RS, W, HEAD_DIM))


def _attention(ri, rc, q4, k4, v4):
    grid_spec = pltpu.PrefetchScalarGridSpec(
        num_scalar_prefetch=2,
        grid=(NUM_HEADS, N_WIN),
        in_specs=[
            pl.BlockSpec((1, RS, W, HEAD_DIM), lambda h, i, *_: (h, i, 0, 0)),
            pl.BlockSpec((1, H, W, HEAD_DIM), lambda h, i, *_: (h, 0, 0, 0)),
            pl.BlockSpec((1, H, W, HEAD_DIM), lambda h, i, *_: (h, 0, 0, 0)),
        ],
        out_specs=pl.BlockSpec((1, RS, W, HEAD_DIM), lambda h, i, *_: (h, i, 0, 0)),
    )
    return pl.pallas_call(
        _attn_body,
        grid_spec=grid_spec,
        out_shape=jax.ShapeDtypeStruct((NUM_HEADS, H, W, HEAD_DIM), jnp.bfloat16),
    )(ri, rc, q4, k4, v4)


RCH = 8


def _out_body(a_ref, vp_ref, wh_ref, wl_ref, w9_ref, lb_ref, b_ref, o_ref,
              buf, sem):
    i = pl.program_id(0)

    def dma(slot, blk):
        return pltpu.make_async_copy(vp_ref.at[pl.ds(blk * RCH, RCH + 2)],
                                     buf.at[slot], sem.at[slot])

    @pl.when(i == 0)
    def _():
        dma(0, 0).start()

    @pl.when(i + 1 < H // RCH)
    def _():
        dma((i + 1) % 2, i + 1).start()

    dma(i % 2, i).wait()
    rows10 = buf[i % 2].astype(jnp.float32)
    lep = jnp.zeros((RCH, W, DIM), jnp.float32) + lb_ref[0]
    for dy in range(3):
        for dx in range(3):
            lep = lep + rows10[dy:dy + RCH, dx:dx + W, :] * w9_ref[dy * 3 + dx]
    l16 = lep.astype(jnp.bfloat16)
    for h in range(RCH):
        acc = jax.lax.dot_general(wl_ref[:], l16[h], (((0,), (1,)), ((), ())),
                                  preferred_element_type=jnp.float32)
        for hd in range(NUM_HEADS):
            acc = acc + jax.lax.dot_general(
                wh_ref[hd], a_ref[hd, h], (((0,), (1,)), ((), ())),
                preferred_element_type=jnp.float32)
        o_ref[h] = acc + b_ref[:]


def _out_proj(attn4, vp, wh, wl, w9, lb, b):
    return pl.pallas_call(
        _out_body,
        grid=(H // RCH,),
        in_specs=[
            pl.BlockSpec((NUM_HEADS, RCH, W, HEAD_DIM), lambda i: (0, i, 0, 0)),
            pl.BlockSpec(memory_space=pl.ANY),
            pl.BlockSpec((NUM_HEADS, HEAD_DIM, DIM), lambda i: (0, 0, 0)),
            pl.BlockSpec((DIM, DIM), lambda i: (0, 0)),
            pl.BlockSpec((9, DIM), lambda i: (0, 0)),
            pl.BlockSpec((1, DIM), lambda i: (0, 0)),
            pl.BlockSpec((DIM, 1), lambda i: (0, 0)),
        ],
        out_specs=pl.BlockSpec((RCH, DIM, W), lambda i: (i, 0, 0)),
        out_shape=jax.ShapeDtypeStruct((H, DIM, W), jnp.float32),
        scratch_shapes=[
            pltpu.VMEM((2, RCH + 2, W + 2, DIM), jnp.bfloat16),
            pltpu.SemaphoreType.DMA((2,)),
        ],
    )(attn4, vp, wh, wl, w9, lb, b)


def kernel(x, qkv_w, qkv_b, lepe_w, lepe_b, out_w, out_b):
    x_t = jnp.transpose(x, (0, 1, 3, 2)).reshape(H, DIM, W)

    q4, k4, v4, v_sp, qr, kr = _qkv_proj(x_t, qkv_w.T, qkv_b.reshape(1, -1))
    ri, rc = _route(qr, kr)

    q4 = q4.reshape(NUM_HEADS, H, W, HEAD_DIM)
    k4 = k4.reshape(NUM_HEADS, H, W, HEAD_DIM)
    v4 = v4.reshape(NUM_HEADS, H, W, HEAD_DIM)
    attn4 = _attention(ri, rc, q4, k4, v4)

    vp = jnp.pad(v_sp.reshape(H, W, DIM), ((1, 15), (1, 1), (0, 0)))

    wT16 = out_w.T.astype(jnp.bfloat16)
    wh = wT16.reshape(NUM_HEADS, HEAD_DIM, DIM)
    out = _out_proj(attn4, vp, wh, wT16, lepe_w.reshape(DIM, 9).T,
                    lepe_b.reshape(1, -1), out_b.reshape(-1, 1))
    return jnp.transpose(out.reshape(1, H, DIM, W), (0, 1, 3, 2))
